# Initial kernel scaffold; baseline (speedup 1.0000x reference)
import jax
import jax.numpy as jnp
from jax.experimental import pallas as pl


def kernel(x, batch):
    raise NotImplementedError("write your pallas kernel here")


# speedup vs baseline: 2.1008x; 2.1008x over previous
import jax
import jax.numpy as jnp
from jax import lax
from jax.experimental import pallas as pl
from jax.experimental.pallas import tpu as pltpu
from jax.experimental.pallas import tpu_sc as plsc

N_ROWS = 320000
N_COLS = 128
N_SEG = 10000
N_SEG_PAD = 10240
CHUNK = 128
N_CHUNKS = N_ROWS // CHUNK
N_WORKERS = 32
STEPS = (N_CHUNKS + N_WORKERS - 1) // N_WORKERS
SEG_SLICE = N_SEG_PAD // 16


def _mesh():
    return plsc.VectorSubcoreMesh(core_axis_name="c", subcore_axis_name="s")


def _k0_body(ids_hbm, outc_hbm, cnts_sh, ids_v, ones_v, czb_v):
    c = lax.axis_index("c")
    s = lax.axis_index("s")
    wid = s * 2 + c

    def init_ones(i, _):
        for j in range(8):
            ones_v[i, pl.ds(j * 16, 16)] = jnp.ones((16,), jnp.float32)
            czb_v[i, pl.ds(j * 16, 16)] = jnp.zeros((16,), jnp.float32)
        return 0
    lax.fori_loop(0, CHUNK, init_ones, 0)

    for jj in range(5):
        pltpu.sync_copy(czb_v, cnts_sh.at[pl.ds(s * SEG_SLICE + jj * CHUNK, CHUNK)])
    plsc.subcore_barrier()

    def step(k, _):
        chunk = k * N_WORKERS + wid

        @pl.when(chunk < N_CHUNKS)
        def _():
            base = chunk * CHUNK
            pltpu.sync_copy(ids_hbm.at[pl.ds(base, CHUNK)], ids_v)
            pltpu.sync_copy(ones_v, cnts_sh.at[ids_v], add=True)
        return 0
    lax.fori_loop(0, STEPS, step, 0)
    plsc.subcore_barrier()

    pltpu.sync_copy(cnts_sh.at[pl.ds(s * SEG_SLICE, SEG_SLICE)],
                    outc_hbm.at[c, pl.ds(s * SEG_SLICE, SEG_SLICE)])


def _k1_body(x_hbm, ids_hbm, outs_hbm, sums_sh, x_v, ids_v):
    c = lax.axis_index("c")
    s = lax.axis_index("s")
    wid = s * 2 + c

    def zrow(i, _):
        for j in range(8):
            x_v[i, pl.ds(j * 16, 16)] = jnp.zeros((16,), jnp.float32)
        return 0
    lax.fori_loop(0, CHUNK, zrow, 0)

    for jj in range(5):
        pltpu.sync_copy(x_v, sums_sh.at[pl.ds(s * SEG_SLICE + jj * CHUNK, CHUNK)])
    plsc.subcore_barrier()

    def step(k, _):
        chunk = k * N_WORKERS + wid

        @pl.when(chunk < N_CHUNKS)
        def _():
            base = chunk * CHUNK
            pltpu.sync_copy(ids_hbm.at[pl.ds(base, CHUNK)], ids_v)
            pltpu.sync_copy(x_hbm.at[pl.ds(base, CHUNK)], x_v)
            pltpu.sync_copy(x_v, sums_sh.at[ids_v], add=True)
        return 0
    lax.fori_loop(0, STEPS, step, 0)
    plsc.subcore_barrier()

    pltpu.sync_copy(sums_sh.at[pl.ds(s * SEG_SLICE, SEG_SLICE)],
                    outs_hbm.at[c, pl.ds(s * SEG_SLICE, SEG_SLICE)])


def _k2_body(s_ref, c_ref, o_ref):
    tot = s_ref[0] + s_ref[1]
    cnt = c_ref[0, :, 0:1] + c_ref[1, :, 0:1]
    o_ref[...] = tot / jnp.maximum(cnt, 1.0)


def _k3_body(x_hbm, ids_hbm, mean_hbm, out_hbm, x_v, m_v, ids_v, sem):
    c = lax.axis_index("c")
    s = lax.axis_index("s")
    wid = s * 2 + c

    def step(k, _):
        chunk = k * N_WORKERS + wid

        @pl.when(chunk < N_CHUNKS)
        def _():
            base = chunk * CHUNK
            pltpu.sync_copy(ids_hbm.at[pl.ds(base, CHUNK)], ids_v)
            pltpu.sync_copy(x_hbm.at[pl.ds(base, CHUNK)], x_v)
            pltpu.async_copy(mean_hbm.at[ids_v], m_v, sem).wait()

            def row(r, _):
                for j in range(8):
                    sl = pl.ds(j * 16, 16)
                    x_v[r, sl] = x_v[r, sl] - m_v[r, sl]
                return 0
            lax.fori_loop(0, CHUNK, row, 0)
            pltpu.sync_copy(x_v, out_hbm.at[pl.ds(base, CHUNK)])
        return 0
    lax.fori_loop(0, STEPS, step, 0)


def kernel(x, batch):
    ids = batch.astype(jnp.int32)

    k0 = pl.kernel(
        _k0_body,
        out_type=jax.ShapeDtypeStruct((2, N_SEG_PAD, N_COLS), jnp.float32),
        mesh=_mesh(),
        scratch_types=[
            pltpu.VMEM_SHARED((N_SEG_PAD, N_COLS), jnp.float32),
            pltpu.VMEM((CHUNK,), jnp.int32),
            pltpu.VMEM((CHUNK, N_COLS), jnp.float32),
            pltpu.VMEM((CHUNK, N_COLS), jnp.float32),
        ],
    )
    part_c = k0(ids)

    k1 = pl.kernel(
        _k1_body,
        out_type=jax.ShapeDtypeStruct((2, N_SEG_PAD, N_COLS), jnp.float32),
        mesh=_mesh(),
        scratch_types=[
            pltpu.VMEM_SHARED((N_SEG_PAD, N_COLS), jnp.float32),
            pltpu.VMEM((CHUNK, N_COLS), jnp.float32),
            pltpu.VMEM((CHUNK,), jnp.int32),
        ],
    )
    part_s = k1(x, ids)

    mean = pl.pallas_call(
        _k2_body,
        out_shape=jax.ShapeDtypeStruct((N_SEG_PAD, N_COLS), jnp.float32),
    )(part_s, part_c)

    k3 = pl.kernel(
        _k3_body,
        out_type=jax.ShapeDtypeStruct((N_ROWS, N_COLS), jnp.float32),
        mesh=_mesh(),
        scratch_types=[
            pltpu.VMEM((CHUNK, N_COLS), jnp.float32),
            pltpu.VMEM((CHUNK, N_COLS), jnp.float32),
            pltpu.VMEM((CHUNK,), jnp.int32),
            pltpu.SemaphoreType.DMA,
        ],
    )
    return k3(x, ids, mean)


# speedup vs baseline: 2.9799x; 1.4184x over previous
import jax
import jax.numpy as jnp
from jax import lax
from jax.experimental import pallas as pl
from jax.experimental.pallas import tpu as pltpu
from jax.experimental.pallas import tpu_sc as plsc

N_ROWS = 320000
N_COLS = 128
N_SEG = 10000
N_SEG_PAD = 10240
CHUNK = 128
N_CHUNKS = N_ROWS // CHUNK
N_WORKERS = 32
STEPS = (N_CHUNKS + N_WORKERS - 1) // N_WORKERS
SEG_SLICE = N_SEG_PAD // 16


def _mesh():
    return plsc.VectorSubcoreMesh(core_axis_name="c", subcore_axis_name="s")


def _k0_body(ids_hbm, outc_hbm, cnts_sh, ids_v, ones_v, czb_v):
    c = lax.axis_index("c")
    s = lax.axis_index("s")
    wid = s * 2 + c

    def init_ones(i, _):
        for j in range(8):
            ones_v[i, pl.ds(j * 16, 16)] = jnp.ones((16,), jnp.float32)
            czb_v[i, pl.ds(j * 16, 16)] = jnp.zeros((16,), jnp.float32)
        return 0
    lax.fori_loop(0, CHUNK, init_ones, 0)

    for jj in range(5):
        pltpu.sync_copy(czb_v, cnts_sh.at[pl.ds(s * SEG_SLICE + jj * CHUNK, CHUNK)])
    plsc.subcore_barrier()

    def step(k, _):
        chunk = k * N_WORKERS + wid

        @pl.when(chunk < N_CHUNKS)
        def _():
            base = chunk * CHUNK
            pltpu.sync_copy(ids_hbm.at[pl.ds(base, CHUNK)], ids_v)
            pltpu.sync_copy(ones_v, cnts_sh.at[ids_v], add=True)
        return 0
    lax.fori_loop(0, STEPS, step, 0)
    plsc.subcore_barrier()

    pltpu.sync_copy(cnts_sh.at[pl.ds(s * SEG_SLICE, SEG_SLICE)],
                    outc_hbm.at[c, pl.ds(s * SEG_SLICE, SEG_SLICE)])


def _k1_body(x_hbm, ids_hbm, outs_hbm, sums_sh, x_v, ids_v):
    c = lax.axis_index("c")
    s = lax.axis_index("s")
    wid = s * 2 + c

    def zrow(i, _):
        for j in range(8):
            x_v[i, pl.ds(j * 16, 16)] = jnp.zeros((16,), jnp.float32)
        return 0
    lax.fori_loop(0, CHUNK, zrow, 0)

    for jj in range(5):
        pltpu.sync_copy(x_v, sums_sh.at[pl.ds(s * SEG_SLICE + jj * CHUNK, CHUNK)])
    plsc.subcore_barrier()

    def step(k, _):
        chunk = k * N_WORKERS + wid

        @pl.when(chunk < N_CHUNKS)
        def _():
            base = chunk * CHUNK
            pltpu.sync_copy(ids_hbm.at[pl.ds(base, CHUNK)], ids_v)
            pltpu.sync_copy(x_hbm.at[pl.ds(base, CHUNK)], x_v)
            pltpu.sync_copy(x_v, sums_sh.at[ids_v], add=True)
        return 0
    lax.fori_loop(0, STEPS, step, 0)
    plsc.subcore_barrier()

    pltpu.sync_copy(sums_sh.at[pl.ds(s * SEG_SLICE, SEG_SLICE)],
                    outs_hbm.at[c, pl.ds(s * SEG_SLICE, SEG_SLICE)])


def _k2_body(s_ref, c_ref, o_ref):
    tot = s_ref[0] + s_ref[1]
    cnt = c_ref[0, :, 0:1] + c_ref[1, :, 0:1]
    o_ref[...] = -(tot / jnp.maximum(cnt, 1.0))


def _k3_body(x_hbm, ids_hbm, nmean_hbm, out_hbm,
             x_v0, x_v1, m_v0, m_v1, ids_v0, ids_v1,
             sem_x0, sem_x1, sem_m0, sem_m1):
    c = lax.axis_index("c")
    s = lax.axis_index("s")
    wid = s * 2 + c

    bufs = ((x_v0, m_v0, ids_v0, sem_x0, sem_m0),
            (x_v1, m_v1, ids_v1, sem_x1, sem_m1))

    def issue(k, b):
        x_v, m_v, ids_v, sem_x, sem_m = bufs[b]
        chunk = k * N_WORKERS + wid

        @pl.when(chunk < N_CHUNKS)
        def _():
            base = chunk * CHUNK
            pltpu.sync_copy(ids_hbm.at[pl.ds(base, CHUNK)], ids_v)
            pltpu.async_copy(x_hbm.at[pl.ds(base, CHUNK)], x_v, sem_x)
            pltpu.async_copy(nmean_hbm.at[ids_v], m_v, sem_m)

    def process(k, b):
        x_v, m_v, ids_v, sem_x, sem_m = bufs[b]
        chunk = k * N_WORKERS + wid

        @pl.when(chunk < N_CHUNKS)
        def _():
            base = chunk * CHUNK
            pltpu.make_async_copy(x_hbm.at[pl.ds(0, CHUNK)], x_v, sem_x).wait()
            pltpu.make_async_copy(x_hbm.at[pl.ds(0, CHUNK)], m_v, sem_m).wait()

            def row(r, _):
                for j in range(8):
                    sl = pl.ds(j * 16, 16)
                    x_v[r, sl] = x_v[r, sl] + m_v[r, sl]
                return 0
            lax.fori_loop(0, CHUNK, row, 0)
            pltpu.sync_copy(x_v, out_hbm.at[pl.ds(base, CHUNK)])

    issue(0, 0)

    def pair(p, _):
        issue(2 * p + 1, 1)
        process(2 * p, 0)
        issue(2 * p + 2, 0)
        process(2 * p + 1, 1)
        return 0
    lax.fori_loop(0, 40, pair, 0)


def kernel(x, batch):
    ids = batch.astype(jnp.int32)

    k0 = pl.kernel(
        _k0_body,
        out_type=jax.ShapeDtypeStruct((2, N_SEG_PAD, N_COLS), jnp.float32),
        mesh=_mesh(),
        scratch_types=[
            pltpu.VMEM_SHARED((N_SEG_PAD, N_COLS), jnp.float32),
            pltpu.VMEM((CHUNK,), jnp.int32),
            pltpu.VMEM((CHUNK, N_COLS), jnp.float32),
            pltpu.VMEM((CHUNK, N_COLS), jnp.float32),
        ],
    )
    part_c = k0(ids)

    k1 = pl.kernel(
        _k1_body,
        out_type=jax.ShapeDtypeStruct((2, N_SEG_PAD, N_COLS), jnp.float32),
        mesh=_mesh(),
        scratch_types=[
            pltpu.VMEM_SHARED((N_SEG_PAD, N_COLS), jnp.float32),
            pltpu.VMEM((CHUNK, N_COLS), jnp.float32),
            pltpu.VMEM((CHUNK,), jnp.int32),
        ],
    )
    part_s = k1(x, ids)

    mean = pl.pallas_call(
        _k2_body,
        out_shape=jax.ShapeDtypeStruct((N_SEG_PAD, N_COLS), jnp.float32),
    )(part_s, part_c)

    k3 = pl.kernel(
        _k3_body,
        out_type=jax.ShapeDtypeStruct((N_ROWS, N_COLS), jnp.float32),
        mesh=_mesh(),
        scratch_types=[
            pltpu.VMEM((CHUNK, N_COLS), jnp.float32),
            pltpu.VMEM((CHUNK, N_COLS), jnp.float32),
            pltpu.VMEM((CHUNK, N_COLS), jnp.float32),
            pltpu.VMEM((CHUNK, N_COLS), jnp.float32),
            pltpu.VMEM((CHUNK,), jnp.int32),
            pltpu.VMEM((CHUNK,), jnp.int32),
            pltpu.SemaphoreType.DMA,
            pltpu.SemaphoreType.DMA,
            pltpu.SemaphoreType.DMA,
            pltpu.SemaphoreType.DMA,
        ],
    )
    return k3(x, ids, mean)


# speedup vs baseline: 3.4730x; 1.1655x over previous
import jax
import jax.numpy as jnp
from jax import lax
from jax.experimental import pallas as pl
from jax.experimental.pallas import tpu as pltpu
from jax.experimental.pallas import tpu_sc as plsc

N_ROWS = 320000
N_COLS = 128
N_SEG = 10000
N_SEG_PAD = 10240
CHUNK = 128
N_CHUNKS = N_ROWS // CHUNK
N_WORKERS = 32
STEPS = (N_CHUNKS + N_WORKERS - 1) // N_WORKERS
SEG_SLICE = N_SEG_PAD // 16


def _mesh():
    return plsc.VectorSubcoreMesh(core_axis_name="c", subcore_axis_name="s")


def _k0_body(ids_hbm, outc_hbm, cnts_sh, ids_v, ids_v1, ones_v, czb_v, sem0, sem1):
    c = lax.axis_index("c")
    s = lax.axis_index("s")
    wid = s * 2 + c

    def init_ones(i, _):
        for j in range(8):
            ones_v[i, pl.ds(j * 16, 16)] = jnp.ones((16,), jnp.float32)
            czb_v[i, pl.ds(j * 16, 16)] = jnp.zeros((16,), jnp.float32)
        return 0
    lax.fori_loop(0, CHUNK, init_ones, 0)

    for jj in range(5):
        pltpu.sync_copy(czb_v, cnts_sh.at[pl.ds(s * SEG_SLICE + jj * CHUNK, CHUNK)])
    plsc.subcore_barrier()

    bufs = ((ids_v, sem0), (ids_v1, sem1))

    def issue(k, b):
        idv, sem = bufs[b]
        chunk = k * N_WORKERS + wid

        @pl.when(chunk < N_CHUNKS)
        def _():
            pltpu.async_copy(ids_hbm.at[pl.ds(chunk * CHUNK, CHUNK)], idv, sem)

    def process(k, b):
        idv, sem = bufs[b]
        chunk = k * N_WORKERS + wid

        @pl.when(chunk < N_CHUNKS)
        def _():
            pltpu.make_async_copy(ids_hbm.at[pl.ds(0, CHUNK)], idv, sem).wait()
            pltpu.sync_copy(ones_v, cnts_sh.at[idv], add=True)

    issue(0, 0)

    def pair(p, _):
        issue(2 * p + 1, 1)
        process(2 * p, 0)
        issue(2 * p + 2, 0)
        process(2 * p + 1, 1)
        return 0
    lax.fori_loop(0, 40, pair, 0)
    plsc.subcore_barrier()

    pltpu.sync_copy(cnts_sh.at[pl.ds(s * SEG_SLICE, SEG_SLICE)],
                    outc_hbm.at[c, pl.ds(s * SEG_SLICE, SEG_SLICE)])


def _k1_body(x_hbm, ids_hbm, outs_hbm, sums_sh,
             x_v0, x_v1, ids_v0, ids_v1, sem_x0, sem_x1):
    c = lax.axis_index("c")
    s = lax.axis_index("s")
    wid = s * 2 + c

    def zrow(i, _):
        for j in range(8):
            x_v0[i, pl.ds(j * 16, 16)] = jnp.zeros((16,), jnp.float32)
        return 0
    lax.fori_loop(0, CHUNK, zrow, 0)

    for jj in range(5):
        pltpu.sync_copy(x_v0, sums_sh.at[pl.ds(s * SEG_SLICE + jj * CHUNK, CHUNK)])
    plsc.subcore_barrier()

    bufs = ((x_v0, ids_v0, sem_x0), (x_v1, ids_v1, sem_x1))

    def issue(k, b):
        x_v, idv, sem = bufs[b]
        chunk = k * N_WORKERS + wid

        @pl.when(chunk < N_CHUNKS)
        def _():
            base = chunk * CHUNK
            pltpu.sync_copy(ids_hbm.at[pl.ds(base, CHUNK)], idv)
            pltpu.async_copy(x_hbm.at[pl.ds(base, CHUNK)], x_v, sem)

    def process(k, b):
        x_v, idv, sem = bufs[b]
        chunk = k * N_WORKERS + wid

        @pl.when(chunk < N_CHUNKS)
        def _():
            pltpu.make_async_copy(x_hbm.at[pl.ds(0, CHUNK)], x_v, sem).wait()
            pltpu.sync_copy(x_v, sums_sh.at[idv], add=True)

    issue(0, 0)

    def pair(p, _):
        issue(2 * p + 1, 1)
        process(2 * p, 0)
        issue(2 * p + 2, 0)
        process(2 * p + 1, 1)
        return 0
    lax.fori_loop(0, 40, pair, 0)
    plsc.subcore_barrier()

    pltpu.sync_copy(sums_sh.at[pl.ds(s * SEG_SLICE, SEG_SLICE)],
                    outs_hbm.at[c, pl.ds(s * SEG_SLICE, SEG_SLICE)])


def _k2_body(s_ref, c_ref, o_ref):
    tot = s_ref[0] + s_ref[1]
    cnt = c_ref[0, :, 0:1] + c_ref[1, :, 0:1]
    o_ref[...] = -(tot / jnp.maximum(cnt, 1.0))


def _k3_body(x_hbm, ids_hbm, nmean_hbm, out_hbm,
             x_v0, x_v1, m_v0, m_v1, ids_v0, ids_v1,
             sem_x0, sem_x1, sem_m0, sem_m1):
    c = lax.axis_index("c")
    s = lax.axis_index("s")
    wid = s * 2 + c

    bufs = ((x_v0, m_v0, ids_v0, sem_x0, sem_m0),
            (x_v1, m_v1, ids_v1, sem_x1, sem_m1))

    def issue(k, b):
        x_v, m_v, ids_v, sem_x, sem_m = bufs[b]
        chunk = k * N_WORKERS + wid

        @pl.when(chunk < N_CHUNKS)
        def _():
            base = chunk * CHUNK
            pltpu.sync_copy(ids_hbm.at[pl.ds(base, CHUNK)], ids_v)
            pltpu.async_copy(x_hbm.at[pl.ds(base, CHUNK)], x_v, sem_x)
            pltpu.async_copy(nmean_hbm.at[ids_v], m_v, sem_m)

    def process(k, b):
        x_v, m_v, ids_v, sem_x, sem_m = bufs[b]
        chunk = k * N_WORKERS + wid

        @pl.when(chunk < N_CHUNKS)
        def _():
            base = chunk * CHUNK
            pltpu.make_async_copy(x_hbm.at[pl.ds(0, CHUNK)], x_v, sem_x).wait()
            pltpu.make_async_copy(x_hbm.at[pl.ds(0, CHUNK)], m_v, sem_m).wait()

            def row(r, _):
                for j in range(8):
                    sl = pl.ds(j * 16, 16)
                    x_v[r, sl] = x_v[r, sl] + m_v[r, sl]
                return 0
            lax.fori_loop(0, CHUNK, row, 0)
            pltpu.sync_copy(x_v, out_hbm.at[pl.ds(base, CHUNK)])

    issue(0, 0)

    def pair(p, _):
        issue(2 * p + 1, 1)
        process(2 * p, 0)
        issue(2 * p + 2, 0)
        process(2 * p + 1, 1)
        return 0
    lax.fori_loop(0, 40, pair, 0)


def kernel(x, batch):
    ids = batch.astype(jnp.int32)

    k0 = pl.kernel(
        _k0_body,
        out_type=jax.ShapeDtypeStruct((2, N_SEG_PAD, N_COLS), jnp.float32),
        mesh=_mesh(),
        scratch_types=[
            pltpu.VMEM_SHARED((N_SEG_PAD, N_COLS), jnp.float32),
            pltpu.VMEM((CHUNK,), jnp.int32),
            pltpu.VMEM((CHUNK,), jnp.int32),
            pltpu.VMEM((CHUNK, N_COLS), jnp.float32),
            pltpu.VMEM((CHUNK, N_COLS), jnp.float32),
            pltpu.SemaphoreType.DMA,
            pltpu.SemaphoreType.DMA,
        ],
    )
    part_c = k0(ids)

    k1 = pl.kernel(
        _k1_body,
        out_type=jax.ShapeDtypeStruct((2, N_SEG_PAD, N_COLS), jnp.float32),
        mesh=_mesh(),
        scratch_types=[
            pltpu.VMEM_SHARED((N_SEG_PAD, N_COLS), jnp.float32),
            pltpu.VMEM((CHUNK, N_COLS), jnp.float32),
            pltpu.VMEM((CHUNK, N_COLS), jnp.float32),
            pltpu.VMEM((CHUNK,), jnp.int32),
            pltpu.VMEM((CHUNK,), jnp.int32),
            pltpu.SemaphoreType.DMA,
            pltpu.SemaphoreType.DMA,
        ],
    )
    part_s = k1(x, ids)

    mean = pl.pallas_call(
        _k2_body,
        out_shape=jax.ShapeDtypeStruct((N_SEG_PAD, N_COLS), jnp.float32),
    )(part_s, part_c)

    k3 = pl.kernel(
        _k3_body,
        out_type=jax.ShapeDtypeStruct((N_ROWS, N_COLS), jnp.float32),
        mesh=_mesh(),
        scratch_types=[
            pltpu.VMEM((CHUNK, N_COLS), jnp.float32),
            pltpu.VMEM((CHUNK, N_COLS), jnp.float32),
            pltpu.VMEM((CHUNK, N_COLS), jnp.float32),
            pltpu.VMEM((CHUNK, N_COLS), jnp.float32),
            pltpu.VMEM((CHUNK,), jnp.int32),
            pltpu.VMEM((CHUNK,), jnp.int32),
            pltpu.SemaphoreType.DMA,
            pltpu.SemaphoreType.DMA,
            pltpu.SemaphoreType.DMA,
            pltpu.SemaphoreType.DMA,
        ],
    )
    return k3(x, ids, mean)
